# Initial kernel scaffold; baseline (speedup 1.0000x reference)
import jax
import jax.numpy as jnp
from jax.experimental import pallas as pl


def kernel(p, x, idx, Wq, bq, Wk, bk, Wv, bv, Wp1, bp1, gp, betap, Wp2, bp2, gw1, betaw1, Ww1, bw1, gw2, betaw2, Ww2, bw2):
    raise NotImplementedError("write your pallas kernel here")


# speedup vs baseline: 3.0960x; 3.0960x over previous
import functools

import jax
import jax.numpy as jnp
from jax import lax
from jax.experimental import pallas as pl
from jax.experimental.pallas import tpu as pltpu
from jax.experimental.pallas import tpu_sc as plsc

N = 100000
C = 32
NS = 16
D = 48

NW = 32
PER_W = (N * NS) // NW
CH = 400
SUB = 80
N_DMA = CH // SUB


def _sc_gather(table, idx_flat):
    mesh = plsc.VectorSubcoreMesh(core_axis_name="c", subcore_axis_name="s")

    @functools.partial(
        pl.kernel,
        mesh=mesh,
        out_type=jax.ShapeDtypeStruct((N * NS, D), jnp.float32),
        scratch_types=[
            pltpu.VMEM((CH,), jnp.int32),
            pltpu.VMEM((CH, D), jnp.float32),
            pltpu.SemaphoreType.DMA,
        ],
        compiler_params=pltpu.CompilerParams(use_tc_tiling_on_sc=False),
    )
    def gather_k(table_hbm, idx_hbm, out_hbm, idx_v, rows_v, sem):
        wid = lax.axis_index("s") * 2 + lax.axis_index("c")
        base = wid * PER_W

        def body(j, carry):
            off = base + j * CH
            pltpu.sync_copy(idx_hbm.at[pl.ds(off, CH)], idx_v)
            copies = []
            for k in range(N_DMA):
                copies.append(
                    pltpu.async_copy(
                        table_hbm.at[idx_v.at[pl.ds(k * SUB, SUB)]],
                        rows_v.at[pl.ds(k * SUB, SUB)],
                        sem,
                    )
                )
            for c in copies:
                c.wait()
            pltpu.sync_copy(rows_v, out_hbm.at[pl.ds(off, CH)])
            return carry

        lax.fori_loop(0, PER_W // CH, body, 0)

    return gather_k(table, idx_flat)


B = 200


def _tc_body(xp_ref, g_ref,
             Wq_ref, bq_ref, Wk_ref, bk_ref, Wv_ref, bv_ref,
             Wp1_ref, bp1_ref, gp_ref, betap_ref, Wp2_ref, bp2_ref,
             gw1_ref, betaw1_ref, Ww1_ref, bw1_ref,
             gw2_ref, betaw2_ref, Ww2_ref, bw2_ref,
             out_ref):
    f32 = jnp.float32
    xb = xp_ref[:, :C]
    pb = xp_ref[:, C:C + 3]
    g = g_ref[:]
    xg = g[:, :C]
    pg = g[:, C:C + 3]

    q = jnp.dot(xb, Wq_ref[:], preferred_element_type=f32) + bq_ref[0]
    kg = jnp.dot(xg, Wk_ref[:], preferred_element_type=f32) + bk_ref[0]
    vg = jnp.dot(xg, Wv_ref[:], preferred_element_type=f32) + bv_ref[0]

    prel = pg - jnp.broadcast_to(pb[:, None, :], (B, NS, 3)).reshape(B * NS, 3)

    pr = jnp.dot(prel, Wp1_ref[:], preferred_element_type=f32) + bp1_ref[0]
    pr = gp_ref[0] * pr + betap_ref[0]
    pr = jnp.maximum(pr, 0.0)
    pr = jnp.dot(pr, Wp2_ref[:], preferred_element_type=f32) + bp2_ref[0]

    qrep = jnp.broadcast_to(q[:, None, :], (B, NS, C)).reshape(B * NS, C)
    w = kg - qrep + pr
    w = gw1_ref[0] * w + betaw1_ref[0]
    w = jnp.maximum(w, 0.0)
    w = jnp.dot(w, Ww1_ref[:], preferred_element_type=f32) + bw1_ref[0]
    w = gw2_ref[0] * w + betaw2_ref[0]
    w = jnp.maximum(w, 0.0)
    w = jnp.dot(w, Ww2_ref[:], preferred_element_type=f32) + bw2_ref[0]

    wt = jnp.concatenate([w] * (C // 4), axis=1).reshape(B, NS, C)
    wt = wt - jnp.max(wt, axis=1, keepdims=True)
    e = jnp.exp(wt)
    e = e / jnp.sum(e, axis=1, keepdims=True)

    vpr = (vg + pr).reshape(B, NS, C)
    out_ref[:] = jnp.sum(vpr * e, axis=1)


def _tc_fused(xp, g, weights):
    def full(a):
        return pl.BlockSpec(a.shape, lambda i: (0,) * a.ndim)

    in_specs = [
        pl.BlockSpec((B, D), lambda i: (i, 0)),
        pl.BlockSpec((B * NS, D), lambda i: (i, 0)),
    ] + [full(a) for a in weights]
    return pl.pallas_call(
        _tc_body,
        grid=(N // B,),
        in_specs=in_specs,
        out_specs=pl.BlockSpec((B, C), lambda i: (i, 0)),
        out_shape=jax.ShapeDtypeStruct((N, C), jnp.float32),
    )(xp, g, *weights)


def kernel(p, x, idx, Wq, bq, Wk, bk, Wv, bv, Wp1, bp1, gp, betap, Wp2, bp2,
           gw1, betaw1, Ww1, bw1, gw2, betaw2, Ww2, bw2):
    xp = jnp.concatenate(
        [x, p, jnp.zeros((N, D - C - 3), jnp.float32)], axis=1)
    g = _sc_gather(xp, idx.reshape(N * NS))
    weights = (
        Wq, bq.reshape(1, C), Wk, bk.reshape(1, C), Wv, bv.reshape(1, C),
        Wp1, bp1.reshape(1, 3), gp.reshape(1, 3), betap.reshape(1, 3),
        Wp2, bp2.reshape(1, C),
        gw1.reshape(1, C), betaw1.reshape(1, C), Ww1, bw1.reshape(1, 4),
        gw2.reshape(1, 4), betaw2.reshape(1, 4), Ww2, bw2.reshape(1, 4),
    )
    return _tc_fused(xp, g, weights)
